# baseline (speedup 1.0000x reference)
import functools

import jax
import jax.numpy as jnp
from jax import lax
from jax.experimental import pallas as pl
from jax.experimental.pallas import tpu as pltpu
from jax.experimental.pallas import tpu_sc as plsc

_BATCH = 16384
_NF = 26
_VOCAB = 100000
_D = 32
_NNUM = 13

_FAT = 128
_RPF = _FAT // _D
_NFAT = _NF * _VOCAB * _D // _FAT

_NC = 2
_NS = 16
_NW = _NC * _NS
_BPW = _BATCH // _NW
_C = 16
_N = _C * _NF
_NCH = _BPW // _C

_mesh = plsc.VectorSubcoreMesh(core_axis_name="c", subcore_axis_name="s")


@functools.partial(
    pl.kernel,
    out_type=jax.ShapeDtypeStruct((_BATCH * _NF * _D,), jnp.float32),
    mesh=_mesh,
    scratch_types=[
        pltpu.VMEM((_BPW * _NF,), jnp.int32),
        pltpu.VMEM((_N,), jnp.int32),
        pltpu.VMEM((_N,), jnp.int32),
        pltpu.VMEM((_N, _FAT), jnp.float32),
        pltpu.VMEM((_N * _D,), jnp.float32),
        pltpu.SemaphoreType.DMA,
    ],
    compiler_params=pltpu.CompilerParams(needs_layout_passes=False),
)
def _gather_kernel(tables_fat, x_cat_flat, emb, xcv, fidxv, qv, gbuf, pbuf, gsem):
    wid = lax.axis_index("s") * _NC + lax.axis_index("c")
    base = wid * _BPW
    lanes = lax.iota(jnp.int32, 16)

    pltpu.sync_copy(x_cat_flat.at[pl.ds(base * _NF, _BPW * _NF)], xcv)

    def chunk_body(k, _):
        def build(g, _):
            pos = g * 16 + lanes
            col = pos % _NF
            v = xcv[pl.ds(k * _N + g * 16, 16)]
            flat = v + col * _VOCAB
            fidxv[pl.ds(g * 16, 16)] = flat >> 2
            qv[pl.ds(g * 16, 16)] = flat & 3
            return 0

        lax.fori_loop(0, _N // 16, build, 0)

        pltpu.async_copy(tables_fat.at[fidxv], gbuf, gsem).wait()

        def extract(g, _):
            r = g * 16 + lanes
            qcol = qv[pl.ds(g * 16, 16)] * _D
            for j in range(_D):
                vals = plsc.load_gather(gbuf, [r, qcol + j])
                plsc.store_scatter(pbuf, [r * _D + j], vals)
            return 0

        lax.fori_loop(0, _N // 16, extract, 0)

        pltpu.sync_copy(
            pbuf, emb.at[pl.ds((base + k * _C) * _NF * _D, _N * _D)]
        )
        return 0

    lax.fori_loop(0, _NCH, chunk_body, 0)


def kernel(x_cat, x_num, tables):
    tables_fat = tables.reshape(_NFAT, _FAT)
    emb = _gather_kernel(tables_fat, x_cat.astype(jnp.int32).reshape(-1))
    return jnp.concatenate([emb.reshape(_BATCH, _NF * _D), x_num], axis=1)


# speedup vs baseline: 1.8668x; 1.8668x over previous
import functools

import jax
import jax.numpy as jnp
from jax import lax
from jax.experimental import pallas as pl
from jax.experimental.pallas import tpu as pltpu
from jax.experimental.pallas import tpu_sc as plsc

_B = 16384
_BH = _B // 2
_NF = 26
_VS = 100000
_V0 = 49920
_V1 = 50048
_VMAIN = _V0 + _V1
_D = 32
_ROWS = _NF * _D
_NSLAB_EMB = _ROWS // 8
_NSLABS = _NSLAB_EMB + 2
_PER_SC = _NSLAB_EMB // 2

_mesh = plsc.VectorSubcoreMesh(core_axis_name="c", subcore_axis_name="s")


@functools.partial(
    pl.kernel,
    out_type=jax.ShapeDtypeStruct((_NSLABS * 8, _B), jnp.float32),
    mesh=_mesh,
    scratch_types=[
        pltpu.MemorySpace.VMEM_SHARED((8, _VMAIN), jnp.float32),
        pltpu.MemorySpace.VMEM_SHARED((8, _B), jnp.float32),
        pltpu.VMEM((_V1,), jnp.float32),
        pltpu.VMEM((8, _D), jnp.float32),
        pltpu.VMEM((_BH,), jnp.int32),
        pltpu.VMEM((_BH,), jnp.float32),
    ],
    compiler_params=pltpu.CompilerParams(needs_layout_passes=False),
)
def _emb_kernel(
    tabT, tab_tail, xcat_flat, xnum_flat, out,
    spm_tab, spm_out, tcol, tailv, idxv, obuf,
):
    cid = lax.axis_index("c")
    sid = lax.axis_index("s")
    r = sid // 2
    h = sid % 2
    lanes = lax.iota(jnp.int32, 16)
    r16 = lanes * 0 + r

    def slab_body(k, _):
        a = cid * _PER_SC + k

        @pl.when(sid == 0)
        def _load_slab():
            pltpu.sync_copy(
                tabT.at[pl.ds(a * 8, 8), pl.ds(0, _V0)],
                spm_tab.at[pl.ds(0, 8), pl.ds(0, _V0)],
            )
            pltpu.sync_copy(
                tabT.at[pl.ds(a * 8, 8), pl.ds(_V0, _V1)],
                spm_tab.at[pl.ds(0, 8), pl.ds(_V0, _V1)],
            )

        plsc.subcore_barrier()

        i = (a * 8 + r) // _D
        pltpu.sync_copy(xcat_flat.at[pl.ds(i * _B + h * _BH, _BH)], idxv)
        pltpu.sync_copy(tab_tail.at[pl.ds(a * 8, 8), :], tailv)

        pltpu.sync_copy(
            spm_tab.at[r, pl.ds(0, _V0)], tcol.at[pl.ds(0, _V0)]
        )

        def gather0(g, _):
            idx16 = idxv[pl.ds(g * 16, 16)]
            obuf[pl.ds(g * 16, 16)] = plsc.load_gather(
                tcol, [jnp.minimum(idx16, _V0 - 1)]
            )
            return 0

        lax.fori_loop(0, _BH // 16, gather0, 0)

        pltpu.sync_copy(spm_tab.at[r, pl.ds(_V0, _V1)], tcol)

        def gather1(g, _):
            idx16 = idxv[pl.ds(g * 16, 16)]
            o0 = obuf[pl.ds(g * 16, 16)]
            rel1 = jnp.minimum(jnp.maximum(idx16 - _V0, 0), _V1 - 1)
            o1 = plsc.load_gather(tcol, [rel1])
            tl = plsc.load_gather(
                tailv, [r16, jnp.maximum(idx16 - _VMAIN, 0)]
            )
            sel = jnp.where(idx16 >= _V0, o1, o0)
            obuf[pl.ds(g * 16, 16)] = jnp.where(idx16 >= _VMAIN, tl, sel)
            return 0

        lax.fori_loop(0, _BH // 16, gather1, 0)

        pltpu.sync_copy(obuf, spm_out.at[r, pl.ds(h * _BH, _BH)])
        plsc.subcore_barrier()

        @pl.when(sid == 0)
        def _store_slab():
            pltpu.sync_copy(spm_out, out.at[pl.ds(a * 8, 8), :])

        plsc.subcore_barrier()
        return 0

    lax.fori_loop(0, _PER_SC, slab_body, 0)

    ax = _NSLAB_EMB + cid
    off = (cid * 8 + r) * _B + h * _BH
    pltpu.sync_copy(xnum_flat.at[pl.ds(off, _BH)], obuf)
    pltpu.sync_copy(obuf, spm_out.at[r, pl.ds(h * _BH, _BH)])
    plsc.subcore_barrier()

    @pl.when(sid == 0)
    def _store_xnum():
        pltpu.sync_copy(spm_out, out.at[pl.ds(ax * 8, 8), :])


def kernel(x_cat, x_num, tables):
    tabT = tables.transpose(0, 2, 1).reshape(_ROWS, _VS)
    tab_tail = tabT[:, _VMAIN:]
    xcat_flat = x_cat.astype(jnp.int32).T.reshape(-1)
    xnum_flat = jnp.pad(x_num.T, ((0, 3), (0, 0))).reshape(-1)
    outT = _emb_kernel(tabT, tab_tail, xcat_flat, xnum_flat)
    return outT[: _ROWS + 13].T


# speedup vs baseline: 2.5429x; 1.3621x over previous
import functools

import jax
import jax.numpy as jnp
from jax import lax
from jax.experimental import pallas as pl
from jax.experimental.pallas import tpu as pltpu
from jax.experimental.pallas import tpu_sc as plsc

_B = 16384
_BH = _B // 2
_NF = 26
_VS = 100000
_V0 = 49920
_V1 = 50048
_VMAIN = _V0 + _V1
_D = 32
_ROWS = _NF * _D
_NSLAB_EMB = _ROWS // 8
_NSLABS = _NSLAB_EMB + 2
_PER_SC = _NSLAB_EMB // 2

_mesh = plsc.VectorSubcoreMesh(core_axis_name="c", subcore_axis_name="s")


@functools.partial(
    pl.kernel,
    out_type=jax.ShapeDtypeStruct((_NSLABS * 8, _B), jnp.float32),
    mesh=_mesh,
    scratch_types=[
        pltpu.MemorySpace.VMEM_SHARED((8, _VMAIN), jnp.float32),
        pltpu.MemorySpace.VMEM_SHARED((8, _B), jnp.float32),
        pltpu.VMEM((_V1,), jnp.float32),
        pltpu.VMEM((8, _D), jnp.float32),
        pltpu.VMEM((_BH,), jnp.int32),
        pltpu.VMEM((_BH,), jnp.float32),
        pltpu.SemaphoreType.DMA,
        pltpu.SemaphoreType.DMA,
        pltpu.SemaphoreType.DMA,
    ],
    compiler_params=pltpu.CompilerParams(needs_layout_passes=False),
)
def _emb_kernel(
    tabT, tab_tail, xcat_flat, xnum_flat, out,
    spm_tab, spm_out, tcol, tailv, idxv, obuf, sem0, sem1, semo,
):
    cid = lax.axis_index("c")
    sid = lax.axis_index("s")
    r = sid // 2
    h = sid % 2
    lanes = lax.iota(jnp.int32, 16)
    r16 = lanes * 0 + r
    base = cid * _PER_SC

    def dma_h0(a, sem):
        return pltpu.make_async_copy(
            tabT.at[pl.ds(a * 8, 8), pl.ds(0, _V0)],
            spm_tab.at[pl.ds(0, 8), pl.ds(0, _V0)],
            sem,
        )

    def dma_h1(a, sem):
        return pltpu.make_async_copy(
            tabT.at[pl.ds(a * 8, 8), pl.ds(_V0, _V1)],
            spm_tab.at[pl.ds(0, 8), pl.ds(_V0, _V1)],
            sem,
        )

    def dma_out(a, sem):
        return pltpu.make_async_copy(
            spm_out, out.at[pl.ds(a * 8, 8), :], sem
        )

    @pl.when(sid == 0)
    def _prologue():
        dma_h0(base, sem0).start()
        dma_h1(base, sem1).start()

    def slab_body(k, _):
        a = base + k

        @pl.when(sid == 0)
        def _wait_in0():
            dma_h0(a, sem0).wait()

            @pl.when(k > 0)
            def _wait_out():
                dma_out(a - 1, semo).wait()

        plsc.subcore_barrier()

        i = (a * 8 + r) // _D
        pltpu.sync_copy(xcat_flat.at[pl.ds(i * _B + h * _BH, _BH)], idxv)
        pltpu.sync_copy(tab_tail.at[pl.ds(a * 8, 8), :], tailv)

        pltpu.sync_copy(spm_tab.at[r, pl.ds(0, _V0)], tcol.at[pl.ds(0, _V0)])
        plsc.subcore_barrier()

        @pl.when(sid == 0)
        def _prefetch0():
            @pl.when(k < _PER_SC - 1)
            def _():
                dma_h0(a + 1, sem0).start()

        def gather0(g, _):
            for j in range(4):
                ds = pl.ds(g * 64 + j * 16, 16)
                idx16 = idxv[ds]
                obuf[ds] = plsc.load_gather(
                    tcol, [jnp.minimum(idx16, _V0 - 1)]
                )
            return 0

        lax.fori_loop(0, _BH // 64, gather0, 0)

        @pl.when(sid == 0)
        def _wait_in1():
            dma_h1(a, sem1).wait()

        plsc.subcore_barrier()

        pltpu.sync_copy(spm_tab.at[r, pl.ds(_V0, _V1)], tcol)
        plsc.subcore_barrier()

        @pl.when(sid == 0)
        def _prefetch1():
            @pl.when(k < _PER_SC - 1)
            def _():
                dma_h1(a + 1, sem1).start()

        def gather1(g, _):
            for j in range(4):
                ds = pl.ds(g * 64 + j * 16, 16)
                idx16 = idxv[ds]
                o0 = obuf[ds]
                rel1 = jnp.minimum(jnp.maximum(idx16 - _V0, 0), _V1 - 1)
                o1 = plsc.load_gather(tcol, [rel1])
                tl = plsc.load_gather(
                    tailv, [r16, jnp.maximum(idx16 - _VMAIN, 0)]
                )
                sel = jnp.where(idx16 >= _V0, o1, o0)
                obuf[ds] = jnp.where(idx16 >= _VMAIN, tl, sel)
            return 0

        lax.fori_loop(0, _BH // 64, gather1, 0)

        pltpu.sync_copy(obuf, spm_out.at[r, pl.ds(h * _BH, _BH)])
        plsc.subcore_barrier()

        @pl.when(sid == 0)
        def _store_slab():
            dma_out(a, semo).start()

        return 0

    lax.fori_loop(0, _PER_SC, slab_body, 0)

    @pl.when(sid == 0)
    def _drain_out():
        dma_out(base + _PER_SC - 1, semo).wait()

    plsc.subcore_barrier()

    ax = _NSLAB_EMB + cid
    off = (cid * 8 + r) * _B + h * _BH
    pltpu.sync_copy(xnum_flat.at[pl.ds(off, _BH)], obuf)
    pltpu.sync_copy(obuf, spm_out.at[r, pl.ds(h * _BH, _BH)])
    plsc.subcore_barrier()

    @pl.when(sid == 0)
    def _store_xnum():
        pltpu.sync_copy(spm_out, out.at[pl.ds(ax * 8, 8), :])


def kernel(x_cat, x_num, tables):
    tabT = tables.transpose(0, 2, 1).reshape(_ROWS, _VS)
    tab_tail = tabT[:, _VMAIN:]
    xcat_flat = x_cat.astype(jnp.int32).T.reshape(-1)
    xnum_flat = jnp.pad(x_num.T, ((0, 3), (0, 0))).reshape(-1)
    outT = _emb_kernel(tabT, tab_tail, xcat_flat, xnum_flat)
    return outT[: _ROWS + 13].T
